# Initial kernel scaffold; baseline (speedup 1.0000x reference)
import jax
import jax.numpy as jnp
from jax.experimental import pallas as pl


def kernel(logits, temperatures, top_ps, min_ps, top_ks, noise):
    raise NotImplementedError("write your pallas kernel here")


# speedup vs baseline: 39.9124x; 39.9124x over previous
import functools

import jax
import jax.numpy as jnp
from jax.experimental import pallas as pl
from jax.experimental.pallas import tpu as pltpu

_B = 64
_V = 100000
_RB = 8
_NBITS = 32


def _f2key(x):
    b = jax.lax.bitcast_convert_type(x, jnp.int32)
    return b ^ ((b >> 31) & jnp.int32(0x7FFFFFFF))


def _key2f(k):
    b = k ^ ((k >> 31) & jnp.int32(0x7FFFFFFF))
    return jax.lax.bitcast_convert_type(b, jnp.float32)


def _sampler_kernel(logits_ref, temp_ref, topp_ref, minp_ref, topk_ref,
                    noise_ref, probs_ref, tok_ref, slp_ref):
    x = logits_ref[...] / temp_ref[...]
    m = jnp.max(x, axis=1, keepdims=True)
    e0 = jnp.exp(x - m)
    z0 = jnp.sum(e0, axis=1, keepdims=True)
    topp_rhs = topp_ref[...] * z0
    topk = topk_ref[...].astype(jnp.float32)

    xmin = jnp.min(x, axis=1, keepdims=True)
    lo0 = _f2key(xmin) - 1
    hi0 = _f2key(m)

    def body(_, carry):
        lo, hi = carry
        mid = (lo >> 1) + (hi >> 1) + (lo & hi & 1)
        midf = _key2f(mid)
        gt = x > midf
        cnt = jnp.sum(jnp.where(gt, 1.0, 0.0), axis=1, keepdims=True)
        mass = jnp.sum(jnp.where(gt, e0, 0.0), axis=1, keepdims=True)
        g = (cnt >= topk) | (mass > topp_rhs)
        lo = jnp.where(g, mid, lo)
        hi = jnp.where(g, hi, mid)
        return lo, hi

    lo, _ = jax.lax.fori_loop(0, _NBITS, body, (lo0, hi0))
    thr = _key2f(lo)

    keep = (x > thr) & (e0 >= minp_ref[...])
    ek = jnp.where(keep, e0, 0.0)
    zf = jnp.sum(ek, axis=1, keepdims=True)
    probs = ek / zf
    probs_ref[...] = probs

    q = -jnp.log(noise_ref[...])
    r = probs / q
    tok = jnp.argmax(r, axis=1).astype(jnp.int32)
    tok_ref[...] = tok.reshape(_RB, 1)

    sel = jax.lax.broadcasted_iota(jnp.int32, (_RB, _V), 1) == tok[:, None]
    p_tok = jnp.sum(jnp.where(sel, probs, 0.0), axis=1, keepdims=True)
    slp_ref[...] = jnp.log(p_tok)


@functools.partial(jax.jit, static_argnames=())
def kernel(logits, temperatures, top_ps, min_ps, top_ks, noise):
    col = lambda a: a.reshape(_B, 1)
    row_spec = pl.BlockSpec((_RB, _V), lambda i: (i, 0))
    sc_spec = pl.BlockSpec((_RB, 1), lambda i: (i, 0))
    probs, tok, slp = pl.pallas_call(
        _sampler_kernel,
        grid=(_B // _RB,),
        in_specs=[row_spec, sc_spec, sc_spec, sc_spec, sc_spec, row_spec],
        out_specs=[row_spec, sc_spec, sc_spec],
        out_shape=[
            jax.ShapeDtypeStruct((_B, _V), jnp.float32),
            jax.ShapeDtypeStruct((_B, 1), jnp.int32),
            jax.ShapeDtypeStruct((_B, 1), jnp.float32),
        ],
        compiler_params=pltpu.CompilerParams(
            dimension_semantics=("arbitrary",),
        ),
    )(logits, col(temperatures), col(top_ps), col(min_ps), col(top_ks), noise)
    return probs, tok.reshape(_B), slp
